# baseline (speedup 1.0000x reference)
import jax
import jax.numpy as jnp
from jax import lax
from jax.experimental import pallas as pl
from jax.experimental.pallas import tpu as pltpu
from jax.experimental.pallas import tpu_sc as plsc

M = 1000000
D = 64
B = 16384

NC = 2
NS = 16
NW = NC * NS
R = 31248
TAIL = M - NW * R
L = 16
MAP = R + TAIL
CR = 112
NCH = R // CR
NBUF = 4
CWCAP = CR + L


def _body(voxel, idx, pixels, out, idx_v, map_v, pk_l, cw_l, cbuf,
          csem, wsem, psem):
    wid = lax.axis_index("s") * NC + lax.axis_index("c")
    last = wid == NW - 1
    lo = pl.multiple_of(wid * R, 8)
    hi = jnp.where(last, M, lo + R)

    idx_stage = pltpu.async_copy(idx, idx_v, csem)
    lane = lax.iota(jnp.int32, L)

    def minit(k, carry):
        map_v[pl.ds(k * L, L)] = jnp.full((L,), -1, jnp.int32)
        return carry

    lax.fori_loop(0, MAP // L, minit, jnp.int32(0))
    idx_stage.wait()

    def p1(c, ptr):
        v = idx_v[pl.ds(c * L, L)]
        m = (v >= lo) & (v < hi)
        pk = jnp.where(m, v - lo, 0) * B + (c * L + lane)
        csum = plsc.cumsum(m.astype(jnp.int32))
        plsc.store_scatter(pk_l, [ptr + csum - 1], pk, mask=m)
        return ptr + csum[L - 1]

    n = lax.fori_loop(0, B // L, p1, jnp.int32(0))
    nch = (n + L - 1) // L

    def fix_step(k, carry):
        m = (k * L + lane) < n
        pk = pk_l[pl.ds(k * L, L)]
        loc = jnp.where(m, lax.shift_right_logical(pk, 14), 0)
        plsc.store_scatter(map_v, [loc], pk, mask=m)
        for _ in range(L - 1):
            w = plsc.load_gather(map_v, [loc], mask=m)
            upd = m & (pk > w)
            plsc.store_scatter(map_v, [loc], pk, mask=upd)
        return carry

    lax.fori_loop(0, nch, fix_step, jnp.int32(0))

    def merge_into(off, cstart, crows):
        cnt = jnp.int32(0)
        for k in range(crows // L):
            w = map_v[pl.ds(cstart + k * L, L)]
            inb = w >= 0
            csum = plsc.cumsum(inb.astype(jnp.int32))
            plsc.store_scatter(cw_l, [cnt + csum - 1], w, mask=inb)
            cnt = cnt + csum[L - 1]

        def apply16(b, carry):
            pk = cw_l[pl.ds(b * L, L)]
            rv = lax.shift_right_logical(pk, 14) - cstart
            iv = pk & (B - 1)
            for j in range(L):
                @pl.when(b * L + j < cnt)
                def _get():
                    src = pixels.at[pl.ds(pl.multiple_of(iv[j] * D, 8), D)]
                    pltpu.async_copy(src, cbuf.at[off + rv[j]], psem)
            for j in range(L):
                @pl.when(b * L + j < cnt)
                def _drain():
                    pltpu.make_async_copy(
                        pixels.at[pl.ds(0, D)], cbuf.at[off], psem).wait()
            return carry

        lax.fori_loop(0, (cnt + L - 1) // L, apply16, jnp.int32(0))

    for b in range(NBUF - 1):
        pltpu.async_copy(voxel.at[pl.ds(pl.multiple_of(lo + b * CR, 8), CR)],
                         cbuf.at[pl.ds(b * CR, CR)], csem)

    def cstep(t, carry):
        cur = (t % NBUF) * CR
        base = pl.multiple_of(lo + t * CR, 8)
        pltpu.make_async_copy(voxel.at[pl.ds(base, CR)],
                              cbuf.at[pl.ds(cur, CR)], csem).wait()

        @pl.when(t > 0)
        def _dr():
            pltpu.make_async_copy(cbuf.at[pl.ds(cur, CR)],
                                  out.at[pl.ds(base, CR)], wsem).wait()

        @pl.when(t + NBUF - 1 < NCH)
        def _pref():
            nb = ((t + NBUF - 1) % NBUF) * CR
            src = voxel.at[
                pl.ds(pl.multiple_of(base + (NBUF - 1) * CR, 8), CR)]
            pltpu.async_copy(src, cbuf.at[pl.ds(nb, CR)], csem)

        merge_into(cur, t * CR, CR)

        pltpu.async_copy(cbuf.at[pl.ds(cur, CR)], out.at[pl.ds(base, CR)],
                         wsem)
        return carry

    lax.fori_loop(0, NCH, cstep, jnp.int32(0))
    pltpu.make_async_copy(cbuf.at[pl.ds(0, CR)], out.at[pl.ds(0, CR)],
                          wsem).wait()

    @pl.when(last)
    def _tail():
        pltpu.sync_copy(voxel.at[pl.ds(M - TAIL, TAIL)],
                        cbuf.at[pl.ds(0, TAIL)])
        merge_into(jnp.int32(0), jnp.int32(R), TAIL)
        pltpu.sync_copy(cbuf.at[pl.ds(0, TAIL)], out.at[pl.ds(M - TAIL, TAIL)])


_scatter = pl.kernel(
    _body,
    out_type=jax.ShapeDtypeStruct((M, D), jnp.float32),
    mesh=plsc.VectorSubcoreMesh(core_axis_name="c", subcore_axis_name="s"),
    compiler_params=pltpu.CompilerParams(needs_layout_passes=False),
    scratch_types=[
        pltpu.VMEM((B,), jnp.int32),
        pltpu.VMEM((MAP,), jnp.int32),
        pltpu.VMEM((B,), jnp.int32),
        pltpu.VMEM((CWCAP,), jnp.int32),
        pltpu.VMEM((NBUF * CR, D), jnp.float32),
        pltpu.SemaphoreType.DMA,
        pltpu.SemaphoreType.DMA,
        pltpu.SemaphoreType.DMA,
    ],
)


@jax.jit
def kernel(voxel, scatter_indices, pixels):
    return _scatter(voxel, scatter_indices.reshape(B), pixels.reshape(B * D))
